# baseline (speedup 1.0000x reference)
import functools

import jax
import jax.numpy as jnp
from jax import lax
from jax.experimental import pallas as pl
from jax.experimental.pallas import tpu as pltpu
from jax.experimental.pallas import tpu_sc as plsc

NC = 2
NS = 16
LANES = 8

CHUNK = 72
NB = 2


def _sc_aggregate(x, src, wexp, dst, n_nodes, d, ept):
  nchunks = ept // CHUNK
  npairs = nchunks // NB
  rows_per_tile = -(-n_nodes // (NS * CHUNK)) * CHUNK
  n_pad = NS * rows_per_tile
  nzfull = rows_per_tile // CHUNK
  zrem = rows_per_tile - nzfull * CHUNK

  mesh = plsc.VectorSubcoreMesh(core_axis_name="c", subcore_axis_name="s")

  @functools.partial(
      pl.kernel,
      out_type=jax.ShapeDtypeStruct((NC, n_pad, d), jnp.float32),
      mesh=mesh,
      scratch_types=[
          pltpu.VMEM_SHARED((n_pad, d), jnp.float32),
          [pltpu.VMEM((CHUNK,), jnp.int32) for _ in range(NB)],
          [pltpu.VMEM((CHUNK, 16), jnp.float32) for _ in range(NB)],
          [pltpu.VMEM((CHUNK,), jnp.int32) for _ in range(NB)],
          [pltpu.VMEM((CHUNK, d), jnp.float32) for _ in range(NB)],
          [pltpu.SemaphoreType.DMA for _ in range(NB)],
          [pltpu.SemaphoreType.DMA for _ in range(NB)],
          [pltpu.SemaphoreType.DMA for _ in range(NB)],
      ],
  )
  def agg(x_hbm, src_hbm, wexp_hbm, dst_hbm, out_hbm,
          acc_sh, src_v, wexp_v, dst_v, rows_v, gsem, isem, ssem):
    c = lax.axis_index("c")
    s = lax.axis_index("s")
    wid = s * NC + c
    tbase = wid * ept

    def zero_row(r, carry):
      for g in range(LANES):
        rows_v[0][r, pl.ds(g * 16, 16)] = jnp.zeros((16,), jnp.float32)
      return carry

    lax.fori_loop(0, CHUNK, zero_row, 0)
    for k in range(nzfull):
      pltpu.sync_copy(rows_v[0],
                      acc_sh.at[pl.ds(s * rows_per_tile + k * CHUNK, CHUNK)])
    if zrem:
      pltpu.sync_copy(
          rows_v[0].at[pl.ds(0, zrem)],
          acc_sh.at[pl.ds(s * rows_per_tile + nzfull * CHUNK, zrem)])
    plsc.subcore_barrier()

    def fire_aux(b, ci):
      pltpu.async_copy(src_hbm.at[pl.ds(tbase + ci * CHUNK, CHUNK)],
                       src_v[b], isem[b])
      pltpu.async_copy(wexp_hbm.at[pl.ds(tbase + ci * CHUNK, CHUNK)],
                       wexp_v[b], isem[b])

    def fire_main(b, ci):
      pltpu.async_copy(x_hbm.at[src_v[b]], rows_v[b], gsem[b])
      pltpu.async_copy(dst_hbm.at[pl.ds(tbase + ci * CHUNK, CHUNK)],
                       dst_v[b], isem[b])

    def wait_aux(b):
      pltpu.make_async_copy(
          src_hbm.at[pl.ds(0, CHUNK)], src_v[b], isem[b]).wait()
      pltpu.make_async_copy(
          wexp_hbm.at[pl.ds(0, CHUNK)], wexp_v[b], isem[b]).wait()

    def wait_main(b):
      pltpu.make_async_copy(x_hbm.at[src_v[b]], rows_v[b], gsem[b]).wait()
      pltpu.make_async_copy(
          dst_hbm.at[pl.ds(0, CHUNK)], dst_v[b], isem[b]).wait()

    def wait_scatter(b):
      pltpu.make_async_copy(rows_v[b], acc_sh.at[dst_v[b]], ssem[b]).wait()

    for b in range(NB):
      fire_aux(b, b)
    for b in range(NB):
      wait_aux(b)
      fire_main(b, b)

    def pair_body(p, carry):
      more = p + 1 < npairs
      for b in range(NB):
        ci = p * NB + b
        wait_main(b)

        def mul_body(e, carry2, b=b):
          wv = wexp_v[b][e]
          for g2 in range(LANES):
            rows_v[b][e, pl.ds(g2 * 16, 16)] = (
                rows_v[b][e, pl.ds(g2 * 16, 16)] * wv)
          return carry2

        lax.fori_loop(0, CHUNK, mul_body, 0)
        pltpu.async_copy(rows_v[b], acc_sh.at[dst_v[b]], ssem[b], add=True)

        @pl.when(more)
        def _(b=b, ci=ci):
          fire_aux(b, ci + NB)
      @pl.when(more)
      def _():
        for b in range(NB):
          ci = p * NB + b
          wait_aux(b)
          wait_scatter(b)
          fire_main(b, ci + NB)
      return carry

    lax.fori_loop(0, npairs, pair_body, 0)
    for b in range(NB):
      wait_scatter(b)
    plsc.subcore_barrier()

    pltpu.sync_copy(acc_sh.at[pl.ds(s * rows_per_tile, rows_per_tile)],
                    out_hbm.at[c, pl.ds(s * rows_per_tile, rows_per_tile)])

  return agg(x, src, wexp, dst)


def _combine_matmul(partials, w, b2, n_nodes, d, u, block_rows):
  def body(p_ref, w_ref, b_ref, o_ref):
    p = p_ref[...]
    acc = p[0] + p[1]
    o_ref[...] = (
        jnp.dot(acc, w_ref[...], preferred_element_type=jnp.float32)
        + b_ref[...]
    )

  grid = (n_nodes // block_rows,)
  return pl.pallas_call(
      body,
      grid=grid,
      in_specs=[
          pl.BlockSpec((NC, block_rows, d), lambda i: (0, i, 0)),
          pl.BlockSpec((d, u), lambda i: (0, 0)),
          pl.BlockSpec((1, u), lambda i: (0, 0)),
      ],
      out_specs=pl.BlockSpec((block_rows, u), lambda i: (i, 0)),
      out_shape=jax.ShapeDtypeStruct((n_nodes, u), jnp.float32),
  )(partials, w, b2)


def kernel(x, edge_index, edge_weight, W, b):
  n_nodes, d = x.shape
  u = W.shape[1]
  e = edge_index.shape[1]

  src = edge_index[0].astype(jnp.int32)
  dst = edge_index[1].astype(jnp.int32)
  w = edge_weight.astype(jnp.float32)

  ntiles = NC * NS
  ept = -(-e // (ntiles * CHUNK * NB)) * CHUNK * NB
  pe = ntiles * ept
  if pe != e:
    pad = pe - e
    src = jnp.concatenate([src, jnp.zeros((pad,), jnp.int32)])
    dst = jnp.concatenate([dst, jnp.zeros((pad,), jnp.int32)])
    w = jnp.concatenate([w, jnp.zeros((pad,), jnp.float32)])
  wexp = jnp.broadcast_to(w[:, None], (pe, 16))
  partials = _sc_aggregate(x, src, wexp, dst, n_nodes, d, ept)
  partials = partials[:, :n_nodes]
  return _combine_matmul(partials, W, b.reshape(1, u), n_nodes, d, u, 1000)


# speedup vs baseline: 1.0132x; 1.0132x over previous
import functools

import jax
import jax.numpy as jnp
from jax import lax
from jax.experimental import pallas as pl
from jax.experimental.pallas import tpu as pltpu
from jax.experimental.pallas import tpu_sc as plsc

NC = 2
NS = 16
LANES = 8

CHUNK = 48
NB = 3


def _sc_aggregate(x, src, wexp, dst, n_nodes, d, ept):
  nchunks = ept // CHUNK
  npairs = nchunks // NB
  rows_per_tile = -(-(-(-n_nodes // NS)) // 8) * 8
  n_pad = NS * rows_per_tile
  nzfull = rows_per_tile // CHUNK
  zrem = rows_per_tile - nzfull * CHUNK

  mesh = plsc.VectorSubcoreMesh(core_axis_name="c", subcore_axis_name="s")

  @functools.partial(
      pl.kernel,
      out_type=jax.ShapeDtypeStruct((NC, n_pad, d), jnp.float32),
      mesh=mesh,
      scratch_types=[
          pltpu.VMEM_SHARED((n_pad, d), jnp.float32),
          [pltpu.VMEM((CHUNK,), jnp.int32) for _ in range(NB)],
          [pltpu.VMEM((CHUNK, 16), jnp.float32) for _ in range(NB)],
          [pltpu.VMEM((CHUNK,), jnp.int32) for _ in range(NB)],
          [pltpu.VMEM((CHUNK, d), jnp.float32) for _ in range(NB)],
          [pltpu.SemaphoreType.DMA for _ in range(NB)],
          [pltpu.SemaphoreType.DMA for _ in range(NB)],
          [pltpu.SemaphoreType.DMA for _ in range(NB)],
      ],
  )
  def agg(x_hbm, src_hbm, wexp_hbm, dst_hbm, out_hbm,
          acc_sh, src_v, wexp_v, dst_v, rows_v, gsem, isem, ssem):
    c = lax.axis_index("c")
    s = lax.axis_index("s")
    wid = s * NC + c
    tbase = wid * ept

    def zero_row(r, carry):
      for g in range(LANES):
        rows_v[0][r, pl.ds(g * 16, 16)] = jnp.zeros((16,), jnp.float32)
      return carry

    lax.fori_loop(0, CHUNK, zero_row, 0)
    for k in range(nzfull):
      pltpu.sync_copy(rows_v[0],
                      acc_sh.at[pl.ds(s * rows_per_tile + k * CHUNK, CHUNK)])
    if zrem:
      pltpu.sync_copy(
          rows_v[0].at[pl.ds(0, zrem)],
          acc_sh.at[pl.ds(s * rows_per_tile + nzfull * CHUNK, zrem)])
    plsc.subcore_barrier()

    def fire_aux(b, ci):
      pltpu.async_copy(src_hbm.at[pl.ds(tbase + ci * CHUNK, CHUNK)],
                       src_v[b], isem[b])
      pltpu.async_copy(wexp_hbm.at[pl.ds(tbase + ci * CHUNK, CHUNK)],
                       wexp_v[b], isem[b])

    def fire_main(b, ci):
      pltpu.async_copy(x_hbm.at[src_v[b]], rows_v[b], gsem[b])
      pltpu.async_copy(dst_hbm.at[pl.ds(tbase + ci * CHUNK, CHUNK)],
                       dst_v[b], isem[b])

    def wait_aux(b):
      pltpu.make_async_copy(
          src_hbm.at[pl.ds(0, CHUNK)], src_v[b], isem[b]).wait()
      pltpu.make_async_copy(
          wexp_hbm.at[pl.ds(0, CHUNK)], wexp_v[b], isem[b]).wait()

    def wait_main(b):
      pltpu.make_async_copy(x_hbm.at[src_v[b]], rows_v[b], gsem[b]).wait()
      pltpu.make_async_copy(
          dst_hbm.at[pl.ds(0, CHUNK)], dst_v[b], isem[b]).wait()

    def wait_scatter(b):
      pltpu.make_async_copy(rows_v[b], acc_sh.at[dst_v[b]], ssem[b]).wait()

    for b in range(NB):
      fire_aux(b, b)
    for b in range(NB):
      wait_aux(b)
      fire_main(b, b)

    def pair_body(p, carry):
      more = p + 1 < npairs
      for b in range(NB):
        ci = p * NB + b
        wait_main(b)

        def mul_body(e, carry2, b=b):
          wv = wexp_v[b][e]
          for g2 in range(LANES):
            rows_v[b][e, pl.ds(g2 * 16, 16)] = (
                rows_v[b][e, pl.ds(g2 * 16, 16)] * wv)
          return carry2

        lax.fori_loop(0, CHUNK, mul_body, 0)
        pltpu.async_copy(rows_v[b], acc_sh.at[dst_v[b]], ssem[b], add=True)

        @pl.when(more)
        def _(b=b, ci=ci):
          fire_aux(b, ci + NB)
      @pl.when(more)
      def _():
        for b in range(NB):
          ci = p * NB + b
          wait_aux(b)
          wait_scatter(b)
          fire_main(b, ci + NB)
      return carry

    lax.fori_loop(0, npairs, pair_body, 0)
    for b in range(NB):
      wait_scatter(b)
    plsc.subcore_barrier()

    pltpu.sync_copy(acc_sh.at[pl.ds(s * rows_per_tile, rows_per_tile)],
                    out_hbm.at[c, pl.ds(s * rows_per_tile, rows_per_tile)])

  return agg(x, src, wexp, dst)


def _combine_matmul(partials, w, b2, n_nodes, d, u, block_rows):
  def body(p_ref, w_ref, b_ref, o_ref):
    p = p_ref[...]
    acc = p[0] + p[1]
    o_ref[...] = (
        jnp.dot(acc, w_ref[...], preferred_element_type=jnp.float32)
        + b_ref[...]
    )

  grid = (n_nodes // block_rows,)
  return pl.pallas_call(
      body,
      grid=grid,
      in_specs=[
          pl.BlockSpec((NC, block_rows, d), lambda i: (0, i, 0)),
          pl.BlockSpec((d, u), lambda i: (0, 0)),
          pl.BlockSpec((1, u), lambda i: (0, 0)),
      ],
      out_specs=pl.BlockSpec((block_rows, u), lambda i: (i, 0)),
      out_shape=jax.ShapeDtypeStruct((n_nodes, u), jnp.float32),
  )(partials, w, b2)


def kernel(x, edge_index, edge_weight, W, b):
  n_nodes, d = x.shape
  u = W.shape[1]
  e = edge_index.shape[1]

  src = edge_index[0].astype(jnp.int32)
  dst = edge_index[1].astype(jnp.int32)
  w = edge_weight.astype(jnp.float32)

  ntiles = NC * NS
  ept = -(-e // (ntiles * CHUNK * NB)) * CHUNK * NB
  pe = ntiles * ept
  if pe != e:
    pad = pe - e
    src = jnp.concatenate([src, jnp.zeros((pad,), jnp.int32)])
    dst = jnp.concatenate([dst, jnp.zeros((pad,), jnp.int32)])
    w = jnp.concatenate([w, jnp.zeros((pad,), jnp.float32)])
  wexp = jnp.broadcast_to(w[:, None], (pe, 16))
  partials = _sc_aggregate(x, src, wexp, dst, n_nodes, d, ept)
  partials = partials[:, :n_nodes]
  return _combine_matmul(partials, W, b.reshape(1, u), n_nodes, d, u, 1000)


# speedup vs baseline: 1.0432x; 1.0296x over previous
import functools

import jax
import jax.numpy as jnp
from jax import lax
from jax.experimental import pallas as pl
from jax.experimental.pallas import tpu as pltpu
from jax.experimental.pallas import tpu_sc as plsc

NC = 2
NS = 16
LANES = 8

CHUNK = 64
NB = 2


def _sc_aggregate(x, src, wexp, dst, n_nodes, d, ept):
  nchunks = ept // CHUNK
  npairs = nchunks // NB
  rows_per_tile = -(-(-(-n_nodes // NS)) // 8) * 8
  n_pad = NS * rows_per_tile
  nzfull = rows_per_tile // CHUNK
  zrem = rows_per_tile - nzfull * CHUNK

  mesh = plsc.VectorSubcoreMesh(core_axis_name="c", subcore_axis_name="s")

  @functools.partial(
      pl.kernel,
      out_type=jax.ShapeDtypeStruct((NC, n_pad, d), jnp.float32),
      mesh=mesh,
      scratch_types=[
          pltpu.VMEM_SHARED((n_pad, d), jnp.float32),
          [pltpu.VMEM((CHUNK,), jnp.int32) for _ in range(NB)],
          [pltpu.VMEM((CHUNK, 16), jnp.float32) for _ in range(NB)],
          [pltpu.VMEM((CHUNK,), jnp.int32) for _ in range(NB)],
          [pltpu.VMEM((CHUNK, d), jnp.float32) for _ in range(NB)],
          [pltpu.SemaphoreType.DMA for _ in range(NB)],
          [pltpu.SemaphoreType.DMA for _ in range(NB)],
          [pltpu.SemaphoreType.DMA for _ in range(NB)],
      ],
  )
  def agg(x_hbm, src_hbm, wexp_hbm, dst_hbm, out_hbm,
          acc_sh, src_v, wexp_v, dst_v, rows_v, gsem, isem, ssem):
    c = lax.axis_index("c")
    s = lax.axis_index("s")
    wid = s * NC + c
    tbase = wid * ept

    def zero_row(r, carry):
      for g in range(LANES):
        rows_v[0][r, pl.ds(g * 16, 16)] = jnp.zeros((16,), jnp.float32)
      return carry

    lax.fori_loop(0, CHUNK, zero_row, 0)
    zcps = []
    for k in range(nzfull):
      zcps.append(pltpu.make_async_copy(
          rows_v[0],
          acc_sh.at[pl.ds(s * rows_per_tile + k * CHUNK, CHUNK)], gsem[0]))
    if zrem:
      zcps.append(pltpu.make_async_copy(
          rows_v[0].at[pl.ds(0, zrem)],
          acc_sh.at[pl.ds(s * rows_per_tile + nzfull * CHUNK, zrem)],
          gsem[0]))
    for cp in zcps:
      cp.start()
    for cp in zcps:
      cp.wait()
    plsc.subcore_barrier()

    def fire_aux(b, ci):
      pltpu.async_copy(src_hbm.at[pl.ds(tbase + ci * CHUNK, CHUNK)],
                       src_v[b], isem[b])
      pltpu.async_copy(wexp_hbm.at[pl.ds(tbase + ci * CHUNK, CHUNK)],
                       wexp_v[b], isem[b])

    def fire_main(b, ci):
      pltpu.async_copy(x_hbm.at[src_v[b]], rows_v[b], gsem[b])
      pltpu.async_copy(dst_hbm.at[pl.ds(tbase + ci * CHUNK, CHUNK)],
                       dst_v[b], isem[b])

    def wait_aux(b):
      pltpu.make_async_copy(
          src_hbm.at[pl.ds(0, CHUNK)], src_v[b], isem[b]).wait()
      pltpu.make_async_copy(
          wexp_hbm.at[pl.ds(0, CHUNK)], wexp_v[b], isem[b]).wait()

    def wait_main(b):
      pltpu.make_async_copy(x_hbm.at[src_v[b]], rows_v[b], gsem[b]).wait()
      pltpu.make_async_copy(
          dst_hbm.at[pl.ds(0, CHUNK)], dst_v[b], isem[b]).wait()

    def wait_scatter(b):
      pltpu.make_async_copy(rows_v[b], acc_sh.at[dst_v[b]], ssem[b]).wait()

    for b in range(NB):
      fire_aux(b, b)
    for b in range(NB):
      wait_aux(b)
      fire_main(b, b)

    def pair_body(p, carry):
      more = p + 1 < npairs
      for b in range(NB):
        ci = p * NB + b
        wait_main(b)

        def mul_body(e, carry2, b=b):
          wv = wexp_v[b][e]
          for g2 in range(LANES):
            rows_v[b][e, pl.ds(g2 * 16, 16)] = (
                rows_v[b][e, pl.ds(g2 * 16, 16)] * wv)
          return carry2

        lax.fori_loop(0, CHUNK, mul_body, 0)
        pltpu.async_copy(rows_v[b], acc_sh.at[dst_v[b]], ssem[b], add=True)

        @pl.when(more)
        def _(b=b, ci=ci):
          fire_aux(b, ci + NB)
      @pl.when(more)
      def _():
        for b in range(NB):
          ci = p * NB + b
          wait_aux(b)
          wait_scatter(b)
          fire_main(b, ci + NB)
      return carry

    lax.fori_loop(0, npairs, pair_body, 0)
    for b in range(NB):
      wait_scatter(b)
    plsc.subcore_barrier()

    pltpu.sync_copy(acc_sh.at[pl.ds(s * rows_per_tile, rows_per_tile)],
                    out_hbm.at[c, pl.ds(s * rows_per_tile, rows_per_tile)])

  return agg(x, src, wexp, dst)


def _combine_matmul(partials, w, b2, n_nodes, d, u, block_rows):
  def body(p_ref, w_ref, b_ref, o_ref):
    p = p_ref[...]
    acc = p[0] + p[1]
    o_ref[...] = (
        jnp.dot(acc, w_ref[...], preferred_element_type=jnp.float32)
        + b_ref[...]
    )

  grid = (n_nodes // block_rows,)
  return pl.pallas_call(
      body,
      grid=grid,
      in_specs=[
          pl.BlockSpec((NC, block_rows, d), lambda i: (0, i, 0)),
          pl.BlockSpec((d, u), lambda i: (0, 0)),
          pl.BlockSpec((1, u), lambda i: (0, 0)),
      ],
      out_specs=pl.BlockSpec((block_rows, u), lambda i: (i, 0)),
      out_shape=jax.ShapeDtypeStruct((n_nodes, u), jnp.float32),
  )(partials, w, b2)


def kernel(x, edge_index, edge_weight, W, b):
  n_nodes, d = x.shape
  u = W.shape[1]
  e = edge_index.shape[1]

  src = edge_index[0].astype(jnp.int32)
  dst = edge_index[1].astype(jnp.int32)
  w = edge_weight.astype(jnp.float32)

  ntiles = NC * NS
  ept = -(-e // (ntiles * CHUNK * NB)) * CHUNK * NB
  pe = ntiles * ept
  if pe != e:
    pad = pe - e
    src = jnp.concatenate([src, jnp.zeros((pad,), jnp.int32)])
    dst = jnp.concatenate([dst, jnp.zeros((pad,), jnp.int32)])
    w = jnp.concatenate([w, jnp.zeros((pad,), jnp.float32)])
  wexp = jnp.broadcast_to(w[:, None], (pe, 16))

  partials = _sc_aggregate(x, src, wexp, dst, n_nodes, d, ept)
  partials = partials[:, :n_nodes]
  return _combine_matmul(partials, W, b.reshape(1, u), n_nodes, d, u, 1000)
